# baseline (speedup 1.0000x reference)
import functools

import jax
import jax.numpy as jnp
from jax import lax
from jax.experimental import pallas as pl
from jax.experimental.pallas import tpu as pltpu
from jax.experimental.pallas import tpu_sc as plsc

K = 30
K_PAD = 32
D_DESC = 48
D_SEM = 32
D_OUT = D_DESC + D_SEM
D_PAD = 128
NUM_CLASSES = 20
QB = 256
CQ = 16
NW = 32
LANES = 16


def _scene_emb_body(feats_ref, emb_ref, fp_ref, out_ref):
    f = feats_ref[...]
    lbl = f[:, 1:2].astype(jnp.int32)
    classes = lax.broadcasted_iota(jnp.int32, (1, NUM_CLASSES + 1), 1)
    onehot = (lbl + 1 == classes).astype(jnp.float32)
    emb = jax.lax.dot_general(
        onehot, emb_ref[...],
        (((1,), (0,)), ((), ())),
        preferred_element_type=jnp.float32)
    desc = f[:, 2:]
    desc = jnp.where(lbl == -1, fp_ref[...], desc)
    pad = jnp.zeros((f.shape[0], D_PAD - D_OUT), jnp.float32)
    out_ref[...] = jnp.concatenate([desc, emb, pad], axis=1)


def _scene_emb(scene_features_flat, emb_table, fp_desc_2d):
    n = scene_features_flat.shape[0]
    rb = 2048
    return pl.pallas_call(
        _scene_emb_body,
        grid=(n // rb,),
        in_specs=[
            pl.BlockSpec((rb, scene_features_flat.shape[1]), lambda i: (i, 0)),
            pl.BlockSpec(emb_table.shape, lambda i: (0, 0)),
            pl.BlockSpec(fp_desc_2d.shape, lambda i: (0, 0)),
        ],
        out_specs=pl.BlockSpec((rb, D_PAD), lambda i: (i, 0)),
        out_shape=jax.ShapeDtypeStruct((n, D_PAD), jnp.float32),
    )(scene_features_flat, emb_table, fp_desc_2d)


DEPTH = 5
LN = 128


def _topk_body(q_ref, s_ref, idx_ref, w_ref):
    b = pl.program_id(0)
    ns = s_ref.shape[2]
    nc = ns // LN
    q = q_ref[0]
    s = s_ref[0]
    nq = q.shape[0]
    dot = jax.lax.dot_general(
        q, s, (((1,), (0,)), ((), ())),
        preferred_element_type=jnp.float32)
    q2 = jnp.sum(q * q, axis=1, keepdims=True)
    s2 = jnp.sum(s * s, axis=0)[None, :]
    d2 = jnp.maximum((q2 + s2) - 2.0 * dot, 0.0)
    d3 = d2.reshape(nq, nc, LN)

    c_iota = lax.broadcasted_iota(jnp.int32, (1, nc, 1), 1)
    l_iota = lax.broadcasted_iota(jnp.int32, (1, 1, LN), 2)
    d_iota = lax.broadcasted_iota(jnp.int32, (1, DEPTH, 1), 1)
    iota_k = lax.broadcasted_iota(jnp.int32, (1, K_PAD), 1)
    inf = jnp.float32(jnp.inf)

    def levels(dm):
        def lv(i, st):
            dc, sv, sp = st
            m = jnp.min(dc, axis=1, keepdims=True)
            p = jnp.min(jnp.where(dc == m, c_iota, nc), axis=1,
                        keepdims=True)
            dc = jnp.where(c_iota == p, inf, dc)
            di = d_iota == i
            sv = jnp.where(di, m, sv)
            sp = jnp.where(di, p, sp)
            return dc, sv, sp
        sv0 = jnp.full((nq, DEPTH, LN), inf, jnp.float32)
        sp0 = jnp.zeros((nq, DEPTH, LN), jnp.int32)
        _, sv, sp = lax.fori_loop(0, DEPTH, lv, (dm, sv0, sp0))
        return sv, sp

    sv, sp = levels(d3)
    nst = DEPTH * LN
    svf = sv.reshape(nq, nst)
    flf = (sp * LN + l_iota).reshape(nq, nst)
    sv_last = sv[:, DEPTH - 1, :]

    idx0 = jnp.zeros((nq, K_PAD), jnp.int32)
    val0 = jnp.zeros((nq, K_PAD), jnp.float32)

    def pop(k, carry):
        svc, idxa, vala = carry
        m = jnp.min(svc, axis=1, keepdims=True)
        eq = svc == m
        fl = jnp.min(jnp.where(eq, flf, ns), axis=1, keepdims=True)
        svc = jnp.where(eq & (flf == fl), inf, svc)
        sel = iota_k == k
        idxa = jnp.where(sel, fl, idxa)
        vala = jnp.where(sel, m, vala)
        return svc, idxa, vala

    _, idxa, vala = lax.fori_loop(0, K, pop, (svf, idx0, val0))

    trigger = jnp.any(sv_last <= vala[:, K - 1:K])
    iota_ns = lax.broadcasted_iota(jnp.int32, (1, ns), 1)

    def brute(args):
        idxa_b, vala_b = args

        def step(k, carry):
            d2c, ia, va = carry
            mb = jnp.min(d2c, axis=1, keepdims=True)
            ab = jnp.min(jnp.where(d2c == mb, iota_ns, ns), axis=1,
                         keepdims=True)
            d2c = jnp.where(iota_ns == ab, inf, d2c)
            sel = iota_k == k
            ia = jnp.where(sel, ab, ia)
            va = jnp.where(sel, mb, va)
            return d2c, ia, va

        _, ia, va = lax.fori_loop(0, K, step, (d2, idx0, val0))
        return ia, va

    def keep(args):
        return args

    idxa, vala = lax.cond(trigger, brute, keep, (idxa, vala))

    idx_ref[0] = idxa[:, :K] + b * ns
    w_ref[0] = jnp.exp(-jnp.sqrt(jnp.maximum(vala[:, :K], 1e-12))) / K


def _topk(query_points, scene_points):
    b, nq, _ = query_points.shape
    ns = scene_points.shape[1]
    scene_t = jnp.swapaxes(scene_points, 1, 2)
    idx, w = pl.pallas_call(
        _topk_body,
        grid=(b, nq // QB),
        in_specs=[
            pl.BlockSpec((1, QB, 3), lambda i, j: (i, j, 0)),
            pl.BlockSpec((1, 3, ns), lambda i, j: (i, 0, 0)),
        ],
        out_specs=[
            pl.BlockSpec((1, QB, K), lambda i, j: (i, j, 0)),
            pl.BlockSpec((1, QB, K), lambda i, j: (i, j, 0)),
        ],
        out_shape=[
            jax.ShapeDtypeStruct((b, nq, K), jnp.int32),
            jax.ShapeDtypeStruct((b, nq, K), jnp.float32),
        ],
        compiler_params=pltpu.CompilerParams(
            vmem_limit_bytes=100 * 1024 * 1024),
    )(query_points, scene_t)
    return idx.reshape(b * nq, K), w.reshape(b * nq, K)


def _bag_sc(scene_emb, idx_flat, w_flat, nq_total):
    qw = nq_total // NW
    mesh = plsc.VectorSubcoreMesh(core_axis_name="c", subcore_axis_name="s",
                                  num_cores=2, num_subcores=16)

    @functools.partial(
        pl.kernel,
        out_type=jax.ShapeDtypeStruct((nq_total, D_PAD), jnp.float32),
        mesh=mesh,
        scratch_types=[
            pltpu.VMEM((CQ * K,), jnp.int32),
            pltpu.VMEM((CQ * K,), jnp.float32),
            pltpu.VMEM((CQ * K, D_PAD), jnp.float32),
            pltpu.VMEM((CQ, D_PAD), jnp.float32),
            pltpu.SemaphoreType.DMA,
        ],
    )
    def bag(emb_hbm, idx_hbm, w_hbm, out_hbm, idx_v, w_v, rows_v, out_v, sem):
        wid = lax.axis_index("s") * 2 + lax.axis_index("c")
        for c in range(qw // CQ):
            base_q = wid * qw + c * CQ
            pltpu.sync_copy(idx_hbm.at[pl.ds(base_q * K, CQ * K)], idx_v)
            pltpu.sync_copy(w_hbm.at[pl.ds(base_q * K, CQ * K)], w_v)
            pltpu.async_copy(emb_hbm.at[idx_v], rows_v, sem).wait()

            def one_query(q, _):
                w0 = w_v[pl.ds(q * K, LANES)]
                w1 = w_v[pl.ds(q * K + K - LANES, LANES)]
                acc = [jnp.zeros((LANES,), jnp.float32)
                       for _ in range(D_OUT // LANES)]
                for k in range(K):
                    wk = w0[k] if k < LANES else w1[k - (K - LANES)]
                    r = q * K + k
                    for j in range(D_OUT // LANES):
                        acc[j] = acc[j] + wk * rows_v[r, pl.ds(j * LANES,
                                                               LANES)]
                for j in range(D_OUT // LANES):
                    out_v[q, pl.ds(j * LANES, LANES)] = acc[j]
                return _

            lax.fori_loop(0, CQ, one_query, 0)
            pltpu.sync_copy(out_v, out_hbm.at[pl.ds(base_q, CQ)])

    return bag(scene_emb, idx_flat, w_flat)


def kernel(query_points, scene_points, scene_features, emb_table, fp_point_desc):
    b, nq, _ = query_points.shape
    ns = scene_points.shape[1]
    nf = scene_features.shape[-1]

    scene_emb = _scene_emb(
        scene_features.reshape(b * ns, nf),
        emb_table,
        fp_point_desc.reshape(1, D_DESC))
    idx, w = _topk(query_points, scene_points)
    out = _bag_sc(scene_emb, idx.reshape(-1), w.reshape(-1), b * nq)
    return out[:, :D_OUT].reshape(b, nq, D_OUT)


# speedup vs baseline: 1.2903x; 1.2903x over previous
import functools

import jax
import jax.numpy as jnp
from jax import lax
from jax.experimental import pallas as pl
from jax.experimental.pallas import tpu as pltpu
from jax.experimental.pallas import tpu_sc as plsc

K = 30
K_PAD = 32
D_DESC = 48
D_SEM = 32
D_OUT = D_DESC + D_SEM
D_PAD = 128
NUM_CLASSES = 20
QB = 128
CQ = 16
NW = 32
LANES = 16


def _scene_emb_body(feats_ref, emb_ref, fp_ref, out_ref):
    f = feats_ref[...]
    lbl = f[:, 1:2].astype(jnp.int32)
    classes = lax.broadcasted_iota(jnp.int32, (1, NUM_CLASSES + 1), 1)
    onehot = (lbl + 1 == classes).astype(jnp.float32)
    emb = jax.lax.dot_general(
        onehot, emb_ref[...],
        (((1,), (0,)), ((), ())),
        preferred_element_type=jnp.float32)
    desc = f[:, 2:]
    desc = jnp.where(lbl == -1, fp_ref[...], desc)
    pad = jnp.zeros((f.shape[0], D_PAD - D_OUT), jnp.float32)
    out_ref[...] = jnp.concatenate([desc, emb, pad], axis=1)


def _scene_emb(scene_features_flat, emb_table, fp_desc_2d):
    n = scene_features_flat.shape[0]
    rb = 2048
    return pl.pallas_call(
        _scene_emb_body,
        grid=(n // rb,),
        in_specs=[
            pl.BlockSpec((rb, scene_features_flat.shape[1]), lambda i: (i, 0)),
            pl.BlockSpec(emb_table.shape, lambda i: (0, 0)),
            pl.BlockSpec(fp_desc_2d.shape, lambda i: (0, 0)),
        ],
        out_specs=pl.BlockSpec((rb, D_PAD), lambda i: (i, 0)),
        out_shape=jax.ShapeDtypeStruct((n, D_PAD), jnp.float32),
    )(scene_features_flat, emb_table, fp_desc_2d)


DEPTH = 5
LN = 128


def _topk_body(q_ref, s_ref, idx_ref, w_ref):
    b = pl.program_id(0)
    ns = s_ref.shape[2]
    nc = ns // LN
    q = q_ref[0]
    s = s_ref[0]
    nq = q.shape[0]
    dot = jax.lax.dot_general(
        q, s, (((1,), (0,)), ((), ())),
        preferred_element_type=jnp.float32)
    q2 = jnp.sum(q * q, axis=1, keepdims=True)
    s2 = jnp.sum(s * s, axis=0)[None, :]
    d2 = jnp.maximum((q2 + s2) - 2.0 * dot, 0.0)
    d3 = d2.reshape(nq, nc, LN)

    c_iota = lax.broadcasted_iota(jnp.int32, (1, nc, 1), 1)
    l_iota = lax.broadcasted_iota(jnp.int32, (1, 1, LN), 2)
    d_iota = lax.broadcasted_iota(jnp.int32, (1, DEPTH, 1), 1)
    iota_k = lax.broadcasted_iota(jnp.int32, (1, K_PAD), 1)
    inf = jnp.float32(jnp.inf)

    def levels(dm):
        def lv(i, st):
            dc, sv, sp = st
            m = jnp.min(dc, axis=1, keepdims=True)
            p = jnp.min(jnp.where(dc == m, c_iota, nc), axis=1,
                        keepdims=True)
            dc = jnp.where(c_iota == p, inf, dc)
            di = d_iota == i
            sv = jnp.where(di, m, sv)
            sp = jnp.where(di, p, sp)
            return dc, sv, sp
        sv0 = jnp.full((nq, DEPTH, LN), inf, jnp.float32)
        sp0 = jnp.zeros((nq, DEPTH, LN), jnp.int32)
        _, sv, sp = lax.fori_loop(0, DEPTH, lv, (dm, sv0, sp0))
        return sv, sp

    sv, sp = levels(d3)
    nst = DEPTH * LN
    svf = sv.reshape(nq, nst)
    flf = (sp * LN + l_iota).reshape(nq, nst)
    sv_last = sv[:, DEPTH - 1, :]

    idx0 = jnp.zeros((nq, K_PAD), jnp.int32)
    val0 = jnp.zeros((nq, K_PAD), jnp.float32)

    def pop(k, carry):
        svc, idxa, vala = carry
        m = jnp.min(svc, axis=1, keepdims=True)
        eq = svc == m
        fl = jnp.min(jnp.where(eq, flf, ns), axis=1, keepdims=True)
        svc = jnp.where(eq & (flf == fl), inf, svc)
        sel = iota_k == k
        idxa = jnp.where(sel, fl, idxa)
        vala = jnp.where(sel, m, vala)
        return svc, idxa, vala

    _, idxa, vala = lax.fori_loop(0, K, pop, (svf, idx0, val0))

    trigger = jnp.any(sv_last <= vala[:, K - 1:K])
    iota_ns = lax.broadcasted_iota(jnp.int32, (1, ns), 1)

    def brute(args):
        idxa_b, vala_b = args

        def step(k, carry):
            d2c, ia, va = carry
            mb = jnp.min(d2c, axis=1, keepdims=True)
            ab = jnp.min(jnp.where(d2c == mb, iota_ns, ns), axis=1,
                         keepdims=True)
            d2c = jnp.where(iota_ns == ab, inf, d2c)
            sel = iota_k == k
            ia = jnp.where(sel, ab, ia)
            va = jnp.where(sel, mb, va)
            return d2c, ia, va

        _, ia, va = lax.fori_loop(0, K, step, (d2, idx0, val0))
        return ia, va

    def keep(args):
        return args

    idxa, vala = lax.cond(trigger, brute, keep, (idxa, vala))

    idx_ref[0] = idxa[:, :K] + b * ns
    w_ref[0] = jnp.exp(-jnp.sqrt(jnp.maximum(vala[:, :K], 1e-12))) / K


def _topk(query_points, scene_points):
    b, nq, _ = query_points.shape
    ns = scene_points.shape[1]
    scene_t = jnp.swapaxes(scene_points, 1, 2)
    idx, w = pl.pallas_call(
        _topk_body,
        grid=(b, nq // QB),
        in_specs=[
            pl.BlockSpec((1, QB, 3), lambda i, j: (i, j, 0)),
            pl.BlockSpec((1, 3, ns), lambda i, j: (i, 0, 0)),
        ],
        out_specs=[
            pl.BlockSpec((1, QB, K), lambda i, j: (i, j, 0)),
            pl.BlockSpec((1, QB, K), lambda i, j: (i, j, 0)),
        ],
        out_shape=[
            jax.ShapeDtypeStruct((b, nq, K), jnp.int32),
            jax.ShapeDtypeStruct((b, nq, K), jnp.float32),
        ],
        compiler_params=pltpu.CompilerParams(
            vmem_limit_bytes=100 * 1024 * 1024),
    )(query_points, scene_t)
    return idx.reshape(b * nq, K), w.reshape(b * nq, K)


def _bag_sc(scene_emb, idx_flat, w_flat, nq_total):
    qw = nq_total // NW
    mesh = plsc.VectorSubcoreMesh(core_axis_name="c", subcore_axis_name="s",
                                  num_cores=2, num_subcores=16)

    @functools.partial(
        pl.kernel,
        out_type=jax.ShapeDtypeStruct((nq_total, D_PAD), jnp.float32),
        mesh=mesh,
        scratch_types=[
            pltpu.VMEM((CQ * K,), jnp.int32),
            pltpu.VMEM((CQ * K,), jnp.float32),
            pltpu.VMEM((CQ * K, D_PAD), jnp.float32),
            pltpu.VMEM((CQ, D_PAD), jnp.float32),
            pltpu.SemaphoreType.DMA,
        ],
    )
    def bag(emb_hbm, idx_hbm, w_hbm, out_hbm, idx_v, w_v, rows_v, out_v, sem):
        wid = lax.axis_index("s") * 2 + lax.axis_index("c")
        for c in range(qw // CQ):
            base_q = wid * qw + c * CQ
            pltpu.sync_copy(idx_hbm.at[pl.ds(base_q * K, CQ * K)], idx_v)
            pltpu.sync_copy(w_hbm.at[pl.ds(base_q * K, CQ * K)], w_v)
            pltpu.async_copy(emb_hbm.at[idx_v], rows_v, sem).wait()

            def one_query(q, _):
                w0 = w_v[pl.ds(q * K, LANES)]
                w1 = w_v[pl.ds(q * K + K - LANES, LANES)]
                acc = [jnp.zeros((LANES,), jnp.float32)
                       for _ in range(D_OUT // LANES)]
                for k in range(K):
                    wk = w0[k] if k < LANES else w1[k - (K - LANES)]
                    r = q * K + k
                    for j in range(D_OUT // LANES):
                        acc[j] = acc[j] + wk * rows_v[r, pl.ds(j * LANES,
                                                               LANES)]
                for j in range(D_OUT // LANES):
                    out_v[q, pl.ds(j * LANES, LANES)] = acc[j]
                return _

            lax.fori_loop(0, CQ, one_query, 0)
            pltpu.sync_copy(out_v, out_hbm.at[pl.ds(base_q, CQ)])

    return bag(scene_emb, idx_flat, w_flat)


def kernel(query_points, scene_points, scene_features, emb_table, fp_point_desc):
    b, nq, _ = query_points.shape
    ns = scene_points.shape[1]
    nf = scene_features.shape[-1]

    scene_emb = _scene_emb(
        scene_features.reshape(b * ns, nf),
        emb_table,
        fp_point_desc.reshape(1, D_DESC))
    idx, w = _topk(query_points, scene_points)
    out = _bag_sc(scene_emb, idx.reshape(-1), w.reshape(-1), b * nq)
    return out[:, :D_OUT].reshape(b, nq, D_OUT)
